# baseline (speedup 1.0000x reference)
import functools
import jax
import jax.numpy as jnp
from jax import lax
from jax.experimental import pallas as pl
from jax.experimental.pallas import tpu as pltpu
from jax.experimental.pallas import tpu_sc as plsc

_B = 4
_C = 96
_H = 224
_W = 224
_N = _H * _W
_P = 8
_BLK = 28
_K = 64
_KP = 32
_TN = 6272
_NT = _N // _TN
_BPAD = 896
_NSUB = 32
_TPS = (_B * _K) // _NSUB


def _im2col(x):
    xp = jnp.pad(x, ((0, 0), (0, 0), (1, 1), (1, 1)))
    sl = [xp[:, ci, dh:dh + _H, dw:dw + _W]
          for ci in range(3) for dh in range(3) for dw in range(3)]
    sl += [jnp.zeros((_B, _H, _W), jnp.float32)] * (_KP - 27)
    col = jnp.stack(sl, axis=-1)
    return col.reshape(_B, _N, _KP)


def _resp_body(col_ref, wt_ref, b_ref, out_ref):
    x = col_ref[0]
    f = jnp.dot(x, wt_ref[...], preferred_element_type=jnp.float32)
    f = jnp.maximum(f + b_ref[...], 0.0)
    out_ref[0] = lax.dot_general(
        jnp.ones((1, _C), jnp.float32), f,
        dimension_numbers=(((1,), (1,)), ((), ())),
        preferred_element_type=jnp.float32)


def _resp(colA, WfT, brow):
    return pl.pallas_call(
        _resp_body,
        grid=(_B * _NT,),
        in_specs=[
            pl.BlockSpec((1, _TN, _KP), lambda g: (g // _NT, g % _NT, 0)),
            pl.BlockSpec((_KP, _C), lambda g: (0, 0)),
            pl.BlockSpec((1, _C), lambda g: (0, 0)),
        ],
        out_specs=pl.BlockSpec((1, 1, _TN), lambda g: (g, 0, 0)),
        out_shape=jax.ShapeDtypeStruct((_B * _NT, 1, _TN), jnp.float32),
    )(colA, WfT, brow)


def _lane_gather(x, idx):
    dn = lax.GatherDimensionNumbers(
        offset_dims=(), collapsed_slice_dims=(0,), start_index_map=(0,))
    return lax.gather(x, idx[:, None], dn, slice_sizes=(1,),
                      mode=lax.GatherScatterMode.PROMISE_IN_BOUNDS)


def _sel_body(resp_hbm, col_hbm, out_hbm, band, rows, outv, sem):
    wid = lax.axis_index("s") * 2 + lax.axis_index("c")
    lanes = lax.iota(jnp.int32, 16)
    b = wid >> 3
    p = wid & 7
    pltpu.sync_copy(resp_hbm.at[wid, 0], band)
    gidx_l = []
    rc_l = []
    for q in range(_P):
        def rowbody(rr, carry, q=q):
            bv, bi = carry
            base = rr * _W + q * _BLK
            v0 = band[pl.ds(base, 16)]
            v1 = band[pl.ds(base + 12, 16)]
            i0 = rr * _BLK + lanes
            i1 = i0 + 12
            c0 = (v0 > bv) | ((v0 == bv) & (i0 < bi))
            bv = jnp.where(c0, v0, bv)
            bi = jnp.where(c0, i0, bi)
            c1 = (v1 > bv) | ((v1 == bv) & (i1 < bi))
            bv = jnp.where(c1, v1, bv)
            bi = jnp.where(c1, i1, bi)
            return bv, bi

        bv, bi = lax.fori_loop(0, _BLK, rowbody,
                               (jnp.full((16,), -jnp.inf, jnp.float32),
                                jnp.full((16,), 1 << 20, jnp.int32)))
        cv, ci = bv, bi
        for s in (8, 4, 2, 1):
            perm = (lanes + s) & 15
            ov = _lane_gather(cv, perm)
            oi = _lane_gather(ci, perm)
            take = (ov > cv) | ((ov == cv) & (oi < ci))
            cv = jnp.where(take, ov, cv)
            ci = jnp.where(take, oi, ci)
        qq = ((ci.astype(jnp.float32) + 0.5) * (1.0 / _BLK)).astype(jnp.int32)
        r = ci - qq * _BLK
        row = p * _BLK + qq
        col = q * _BLK + r
        gidx_l.append(b * _N + row * _W + col)
        rc_l.append(jnp.where(lanes == 0, row.astype(jnp.float32),
                              jnp.where(lanes == 1, col.astype(jnp.float32),
                                        0.0)))
    copies = [pltpu.make_async_copy(col_hbm.at[gidx_l[q][0]], rows.at[q], sem)
              for q in range(_P)]
    for c in copies:
        c.start()
    for c in copies:
        c.wait()
    zero = jnp.zeros((16,), jnp.float32)
    for q in range(_P):
        outv[q, pl.ds(0, 16)] = rows[q, pl.ds(0, 16)]
        outv[q, pl.ds(16, 16)] = rows[q, pl.ds(16, 16)]
        outv[q, pl.ds(32, 16)] = rc_l[q]
        for c in range(3, 8):
            outv[q, pl.ds(c * 16, 16)] = zero
    pltpu.sync_copy(outv, out_hbm.at[pl.ds(wid * _TPS, _TPS)])


def _select_gather(resp3, col_flat):
    mesh = plsc.VectorSubcoreMesh(core_axis_name="c", subcore_axis_name="s")
    f = pl.kernel(
        _sel_body,
        mesh=mesh,
        out_type=jax.ShapeDtypeStruct((_B * _K, 128), jnp.float32),
        scratch_types=[
            pltpu.VMEM((_TN,), jnp.float32),
            pltpu.VMEM((_TPS, _KP), jnp.float32),
            pltpu.VMEM((_TPS, 128), jnp.float32),
            pltpu.SemaphoreType.DMA,
        ],
        compiler_params=pltpu.CompilerParams(
            use_tc_tiling_on_sc=True, needs_layout_passes=False),
    )
    return f(resp3, col_flat)


def _dist_body(col_ref, pT_ref, rA_ref, cA_ref, wt_ref, w96_ref, brow_ref,
               bcol_ref, dr_ref, dc_ref, mv_ref, descT, nA, rmin, ridx):
    nt = pl.program_id(1)

    @pl.when(nt == 0)
    def _():
        d = jnp.dot(w96_ref[...], pT_ref[0], preferred_element_type=jnp.float32)
        d = jnp.maximum(d + bcol_ref[...], 0.0)
        descT[...] = d
        nA[...] = jnp.sum(d * d, axis=0, keepdims=True)
        rmin[...] = jnp.full((1, _K), jnp.inf, jnp.float32)
        ridx[...] = jnp.zeros((1, _K), jnp.int32)

    x = col_ref[0]
    f = jnp.dot(x, wt_ref[...], preferred_element_type=jnp.float32)
    f = jnp.maximum(f + brow_ref[...], 0.0)
    dots = jnp.dot(f, descT[...], preferred_element_type=jnp.float32)
    nb = jnp.sum(f * f, axis=1, keepdims=True)
    dist = nb - 2.0 * dots
    tmin = jnp.min(dist, axis=0, keepdims=True)
    ii = lax.broadcasted_iota(jnp.int32, (_TN, _K), 0)
    targ = jnp.min(jnp.where(dist == tmin, ii, jnp.int32(_TN)),
                   axis=0, keepdims=True)
    better = tmin < rmin[...]
    ridx[...] = jnp.where(better, targ + nt * _TN, ridx[...])
    rmin[...] = jnp.where(better, tmin, rmin[...])

    @pl.when(nt == _NT - 1)
    def _():
        idx = ridx[...]
        rB = (idx // _W).astype(jnp.float32)
        cB = (idx % _W).astype(jnp.float32)
        dr_ref[0] = rA_ref[0] - rB
        dc_ref[0] = cA_ref[0] - cB
        mv_ref[0] = rmin[...] + nA[...]


def _dist(colB, pT, rA, cA, WfT, W96, brow, bcol):
    out3 = [jax.ShapeDtypeStruct((_B, 1, _K), jnp.float32)] * 3
    return pl.pallas_call(
        _dist_body,
        grid=(_B, _NT),
        in_specs=[
            pl.BlockSpec((1, _TN, _KP), lambda b, n: (b, n, 0)),
            pl.BlockSpec((1, _KP, _K), lambda b, n: (b, 0, 0)),
            pl.BlockSpec((1, 1, _K), lambda b, n: (b, 0, 0)),
            pl.BlockSpec((1, 1, _K), lambda b, n: (b, 0, 0)),
            pl.BlockSpec((_KP, _C), lambda b, n: (0, 0)),
            pl.BlockSpec((_C, _KP), lambda b, n: (0, 0)),
            pl.BlockSpec((1, _C), lambda b, n: (0, 0)),
            pl.BlockSpec((_C, 1), lambda b, n: (0, 0)),
        ],
        out_specs=[pl.BlockSpec((1, 1, _K), lambda b, n: (b, 0, 0))] * 3,
        out_shape=out3,
        scratch_shapes=[
            pltpu.VMEM((_C, _K), jnp.float32),
            pltpu.VMEM((1, _K), jnp.float32),
            pltpu.VMEM((1, _K), jnp.float32),
            pltpu.VMEM((1, _K), jnp.int32),
        ],
        compiler_params=pltpu.CompilerParams(
            dimension_semantics=("arbitrary", "arbitrary")),
    )(colB, pT, rA, cA, WfT, W96, brow, bcol)


@jax.jit
def kernel(xA, xB, Wc, bc):
    colA = _im2col(xA)
    colB = _im2col(xB)
    Wf = Wc.reshape(_C, 27)
    W96 = jnp.pad(Wf, ((0, 0), (0, _KP - 27)))
    WfT = W96.T
    brow = bc.reshape(1, _C)
    bcol = bc.reshape(_C, 1)

    resp3 = _resp(colA, WfT, brow)
    sel = _select_gather(resp3, colA.reshape(_B * _N, _KP))
    po = sel.reshape(_B, _K, 128)
    pT = po[:, :, :_KP].transpose(0, 2, 1)
    rA = po[:, :, 32].reshape(_B, 1, _K)
    cA = po[:, :, 33].reshape(_B, 1, _K)
    dr, dc, mv = _dist(colB, pT, rA, cA, WfT, W96, brow, bcol)
    return jnp.stack([dr[:, 0, :], dc[:, 0, :], mv[:, 0, :]], axis=-1)


# speedup vs baseline: 10.4521x; 10.4521x over previous
import jax
import jax.numpy as jnp
from jax import lax
from jax.experimental import pallas as pl
from jax.experimental.pallas import tpu as pltpu
from jax.experimental.pallas import tpu_sc as plsc

_B = 4
_C = 96
_H = 224
_W = 224
_N = _H * _W
_P = 8
_BLK = 28
_K = 64
_KP = 32
_CP = 8
_WP = 226
_NR = 228
_TNP = _BLK * _WP
_NT = _H // _BLK
_NB = _B * _NT
_WLEN = 6784
_TPS = 8


def _nhwc8(x):
    y = jnp.zeros((_B, _NR, _WP, _CP), x.dtype)
    y = y.at[:, 2:226, 1:225, :3].set(x.transpose(0, 2, 3, 1))
    return y.reshape(_B, _NR * _WP, _CP)


def _wins(xt):
    return jnp.stack([
        xt[b, (p * _BLK + 1) * _WP - 1:(p * _BLK + 1) * _WP - 1 + _WLEN]
        for b in range(_B) for p in range(_NT)])


def _conv_f(win_ref, wsh_ref, brow_ref):
    acc = jnp.dot(win_ref[0, pl.ds(0, _TNP), :], wsh_ref[0],
                  preferred_element_type=jnp.float32)
    for s in range(1, 9):
        off = (s // 3) * _WP + s % 3
        acc += jnp.dot(win_ref[0, pl.ds(off, _TNP), :], wsh_ref[s],
                       preferred_element_type=jnp.float32)
    return jnp.maximum(acc + brow_ref[...], 0.0)


def _resp_body(win_ref, wsh_ref, b_ref, out_ref):
    f = _conv_f(win_ref, wsh_ref, b_ref)
    out_ref[0] = lax.dot_general(
        jnp.ones((1, _C), jnp.float32), f,
        dimension_numbers=(((1,), (1,)), ((), ())),
        preferred_element_type=jnp.float32)


def _resp(xtwA, Wsh, brow):
    return pl.pallas_call(
        _resp_body,
        grid=(_NB,),
        in_specs=[
            pl.BlockSpec((1, _WLEN, _CP), lambda g: (g, 0, 0)),
            pl.BlockSpec((9, _CP, _C), lambda g: (0, 0, 0)),
            pl.BlockSpec((1, _C), lambda g: (0, 0)),
        ],
        out_specs=pl.BlockSpec((1, 1, _TNP), lambda g: (g, 0, 0)),
        out_shape=jax.ShapeDtypeStruct((_NB, 1, _TNP), jnp.float32),
    )(xtwA, Wsh, brow)


def _lane_gather(x, idx):
    dn = lax.GatherDimensionNumbers(
        offset_dims=(), collapsed_slice_dims=(0,), start_index_map=(0,))
    return lax.gather(x, idx[:, None], dn, slice_sizes=(1,),
                      mode=lax.GatherScatterMode.PROMISE_IN_BOUNDS)


def _fdiv(x, d):
    return ((x.astype(jnp.float32) + 0.5) * (1.0 / d)).astype(jnp.int32)


def _sel_body(resp_hbm, xt_hbm, out_hbm, band, winv, outv, sem):
    wid = lax.axis_index("s") * 2 + lax.axis_index("c")
    lanes = lax.iota(jnp.int32, 16)
    b = wid >> 3
    p = wid & 7
    pltpu.sync_copy(resp_hbm.at[wid, 0], band)
    rc_l = []
    sub_l = []
    copies = []
    for q in range(_P):
        def rowbody(rr, carry, q=q):
            bv, bi = carry
            base = rr * _WP + q * _BLK + 1
            v0 = band[pl.ds(base, 16)]
            v1 = band[pl.ds(base + 12, 16)]
            i0 = rr * _BLK + lanes
            i1 = i0 + 12
            c0 = (v0 > bv) | ((v0 == bv) & (i0 < bi))
            bv = jnp.where(c0, v0, bv)
            bi = jnp.where(c0, i0, bi)
            c1 = (v1 > bv) | ((v1 == bv) & (i1 < bi))
            bv = jnp.where(c1, v1, bv)
            bi = jnp.where(c1, i1, bi)
            return bv, bi

        bv, bi = lax.fori_loop(0, _BLK, rowbody,
                               (jnp.full((16,), -jnp.inf, jnp.float32),
                                jnp.full((16,), 1 << 20, jnp.int32)))
        cv, ci = bv, bi
        for s in (8, 4, 2, 1):
            perm = (lanes + s) & 15
            ov = _lane_gather(cv, perm)
            oi = _lane_gather(ci, perm)
            take = (ov > cv) | ((ov == cv) & (oi < ci))
            cv = jnp.where(take, ov, cv)
            ci = jnp.where(take, oi, ci)
        qq = _fdiv(ci, _BLK)
        r = ci - qq * _BLK
        row = p * _BLK + qq
        col = q * _BLK + r
        rc_l.append(jnp.where(lanes == 0, row.astype(jnp.float32),
                              jnp.where(lanes == 1, col.astype(jnp.float32),
                                        0.0)))
        row0 = row[0]
        col0 = col[0]
        subs = []
        for dh in range(3):
            start = (row0 + dh + 1) * _WP + col0
            start8 = pl.multiple_of((start >> 3) << 3, 8)
            subs.append(start - start8)
            copies.append(pltpu.make_async_copy(
                xt_hbm.at[b, pl.ds(start8, 16)], winv.at[q, dh], sem))
        sub_l.append(subs)
    for c in copies:
        c.start()
    for c in copies:
        c.wait()
    l1 = lanes + 16
    ci0 = _fdiv(lanes, 9)
    rem0 = lanes - 9 * ci0
    dh0 = _fdiv(rem0, 3)
    dw0 = rem0 - 3 * dh0
    ci1 = _fdiv(l1, 9)
    rem1 = l1 - 9 * ci1
    dh1 = _fdiv(rem1, 3)
    dw1 = rem1 - 3 * dh1
    zero = jnp.zeros((16,), jnp.float32)
    for q in range(_P):
        qv = jnp.full((16,), q, jnp.int32)
        s0, s1, s2 = sub_l[q]
        sub0 = jnp.where(dh0 == 0, s0, jnp.where(dh0 == 1, s1, s2))
        sub1 = jnp.where(dh1 == 0, s0, jnp.where(dh1 == 1, s1, s2))
        outv[q, pl.ds(0, 16)] = plsc.load_gather(
            winv, [qv, dh0, sub0 + dw0, ci0])
        outv[q, pl.ds(16, 16)] = plsc.load_gather(
            winv, [qv, dh1, sub1 + dw1, ci1])
        outv[q, pl.ds(32, 16)] = rc_l[q]
        for c in range(3, 8):
            outv[q, pl.ds(c * 16, 16)] = zero
    pltpu.sync_copy(outv, out_hbm.at[pl.ds(wid * _TPS, _TPS)])


def _select_gather(resp3, xtA):
    mesh = plsc.VectorSubcoreMesh(core_axis_name="c", subcore_axis_name="s")
    f = pl.kernel(
        _sel_body,
        mesh=mesh,
        out_type=jax.ShapeDtypeStruct((_B * _K, 128), jnp.float32),
        scratch_types=[
            pltpu.VMEM((_TNP,), jnp.float32),
            pltpu.VMEM((_TPS, 3, 16, _CP), jnp.float32),
            pltpu.VMEM((_TPS, 128), jnp.float32),
            pltpu.SemaphoreType.DMA,
        ],
        compiler_params=pltpu.CompilerParams(
            use_tc_tiling_on_sc=True, needs_layout_passes=False),
    )
    return f(resp3, xtA)


def _dist_body(win_ref, pT_ref, rA_ref, cA_ref, wsh_ref, w96_ref, brow_ref,
               bcol_ref, pen_ref, dr_ref, dc_ref, mv_ref, descT, nA, rmin,
               ridx):
    nt = pl.program_id(1)

    @pl.when(nt == 0)
    def _():
        d = jnp.dot(w96_ref[...], pT_ref[0], preferred_element_type=jnp.float32)
        d = jnp.maximum(d + bcol_ref[...], 0.0)
        descT[...] = d
        nA[...] = jnp.sum(d * d, axis=0, keepdims=True)
        rmin[...] = jnp.full((1, _K), jnp.inf, jnp.float32)
        ridx[...] = jnp.zeros((1, _K), jnp.int32)

    f = _conv_f(win_ref, wsh_ref, brow_ref)
    dots = jnp.dot(f, descT[...], preferred_element_type=jnp.float32)
    nb = jnp.sum(f * f, axis=1, keepdims=True)
    dist = nb - 2.0 * dots + pen_ref[...]
    tmin = jnp.min(dist, axis=0, keepdims=True)
    ii = lax.broadcasted_iota(jnp.int32, (_TNP, _K), 0)
    targ = jnp.min(jnp.where(dist == tmin, ii, jnp.int32(_TNP)),
                   axis=0, keepdims=True)
    better = tmin < rmin[...]
    ridx[...] = jnp.where(better, targ + nt * _TNP, ridx[...])
    rmin[...] = jnp.where(better, tmin, rmin[...])

    @pl.when(nt == _NT - 1)
    def _():
        idx = ridx[...]
        g = idx % _TNP
        rB = ((idx // _TNP) * _BLK + g // _WP).astype(jnp.float32)
        cB = (g % _WP - 1).astype(jnp.float32)
        dr_ref[0] = rA_ref[0] - rB
        dc_ref[0] = cA_ref[0] - cB
        mv_ref[0] = rmin[...] + nA[...]


def _dist(xtwB, pT, rA, cA, Wsh, W96, brow, bcol, pen):
    out3 = [jax.ShapeDtypeStruct((_B, 1, _K), jnp.float32)] * 3
    return pl.pallas_call(
        _dist_body,
        grid=(_B, _NT),
        in_specs=[
            pl.BlockSpec((1, _WLEN, _CP), lambda b, n: (b * _NT + n, 0, 0)),
            pl.BlockSpec((1, _KP, _K), lambda b, n: (b, 0, 0)),
            pl.BlockSpec((1, 1, _K), lambda b, n: (b, 0, 0)),
            pl.BlockSpec((1, 1, _K), lambda b, n: (b, 0, 0)),
            pl.BlockSpec((9, _CP, _C), lambda b, n: (0, 0, 0)),
            pl.BlockSpec((_C, _KP), lambda b, n: (0, 0)),
            pl.BlockSpec((1, _C), lambda b, n: (0, 0)),
            pl.BlockSpec((_C, 1), lambda b, n: (0, 0)),
            pl.BlockSpec((_TNP, 1), lambda b, n: (0, 0)),
        ],
        out_specs=[pl.BlockSpec((1, 1, _K), lambda b, n: (b, 0, 0))] * 3,
        out_shape=out3,
        scratch_shapes=[
            pltpu.VMEM((_C, _K), jnp.float32),
            pltpu.VMEM((1, _K), jnp.float32),
            pltpu.VMEM((1, _K), jnp.float32),
            pltpu.VMEM((1, _K), jnp.int32),
        ],
        compiler_params=pltpu.CompilerParams(
            dimension_semantics=("arbitrary", "arbitrary")),
    )(xtwB, pT, rA, cA, Wsh, W96, brow, bcol, pen)


@jax.jit
def kernel(xA, xB, Wc, bc):
    xtA = _nhwc8(xA)
    xtB = _nhwc8(xB)
    xtwA = _wins(xtA)
    xtwB = _wins(xtB)
    Wf = Wc.reshape(_C, 27)
    Wsh = jnp.pad(Wc.transpose(2, 3, 1, 0), ((0, 0), (0, 0), (0, _CP - 3),
                                             (0, 0))).reshape(9, _CP, _C)
    W96 = jnp.pad(Wf, ((0, 0), (0, _KP - 27)))
    brow = bc.reshape(1, _C)
    bcol = bc.reshape(_C, 1)
    seam = (jnp.arange(_TNP, dtype=jnp.int32) % _WP)
    pen = jnp.where((seam == 0) | (seam == _WP - 1), 1e30, 0.0)
    pen = pen.astype(jnp.float32).reshape(_TNP, 1)

    resp3 = _resp(xtwA, Wsh, brow)
    sel = _select_gather(resp3, xtA)
    po = sel.reshape(_B, _K, 128)
    pT = po[:, :, :_KP].transpose(0, 2, 1)
    rA = po[:, :, 32].reshape(_B, 1, _K)
    cA = po[:, :, 33].reshape(_B, 1, _K)
    dr, dc, mv = _dist(xtwB, pT, rA, cA, Wsh, W96, brow, bcol, pen)
    return jnp.stack([dr[:, 0, :], dc[:, 0, :], mv[:, 0, :]], axis=-1)


# speedup vs baseline: 13.3411x; 1.2764x over previous
import jax
import jax.numpy as jnp
from jax import lax
from jax.experimental import pallas as pl
from jax.experimental.pallas import tpu as pltpu
from jax.experimental.pallas import tpu_sc as plsc

_B = 4
_C = 96
_H = 224
_W = 224
_N = _H * _W
_P = 8
_BLK = 28
_K = 64
_KP = 32
_CP = 8
_WP = 226
_NR = 228
_TN = _BLK * _W
_NT = _H // _BLK
_NB = _B * _NT
_TPS = 8


def _im2col(x):
    xp = jnp.pad(x, ((0, 0), (0, 0), (1, 1), (1, 1)))
    sl = [xp[:, :, dh:dh + _H, dw:dw + _W].reshape(_B, 3, _N)
          for dh in range(3) for dw in range(3)]
    sl.append(jnp.zeros((_B, _KP - 27, _N), jnp.float32))
    return jnp.concatenate(sl, axis=1)


def _nhwc8(x):
    y = jnp.zeros((_B, _NR, _WP, _CP), x.dtype)
    y = y.at[:, 2:226, 1:225, :3].set(x.transpose(0, 2, 3, 1))
    return y.reshape(_B, _NR * _WP, _CP)


def _resp_body(col_ref, wc_ref, b_ref, out_ref):
    f = jnp.dot(wc_ref[...], col_ref[0], preferred_element_type=jnp.float32)
    f = jnp.maximum(f + b_ref[...], 0.0)
    out_ref[0] = jnp.dot(jnp.ones((1, _C), jnp.float32), f,
                         preferred_element_type=jnp.float32)


def _resp(colA, Wcat, bcol):
    return pl.pallas_call(
        _resp_body,
        grid=(_NB,),
        in_specs=[
            pl.BlockSpec((1, _KP, _TN), lambda g: (g // _NT, 0, g % _NT)),
            pl.BlockSpec((_C, _KP), lambda g: (0, 0)),
            pl.BlockSpec((_C, 1), lambda g: (0, 0)),
        ],
        out_specs=pl.BlockSpec((1, 1, _TN), lambda g: (g, 0, 0)),
        out_shape=jax.ShapeDtypeStruct((_NB, 1, _TN), jnp.float32),
    )(colA, Wcat, bcol)


def _lane_gather(x, idx):
    dn = lax.GatherDimensionNumbers(
        offset_dims=(), collapsed_slice_dims=(0,), start_index_map=(0,))
    return lax.gather(x, idx[:, None], dn, slice_sizes=(1,),
                      mode=lax.GatherScatterMode.PROMISE_IN_BOUNDS)


def _fdiv(x, d):
    return ((x.astype(jnp.float32) + 0.5) * (1.0 / d)).astype(jnp.int32)


def _sel_body(resp_hbm, xt_hbm, out_hbm, band, winv, outv, sem):
    wid = lax.axis_index("s") * 2 + lax.axis_index("c")
    lanes = lax.iota(jnp.int32, 16)
    b = wid >> 3
    p = wid & 7
    pltpu.sync_copy(resp_hbm.at[wid, 0], band)
    rc_l = []
    sub_l = []
    copies = []
    for q in range(_P):
        def rowbody(rr, carry, q=q):
            bv, bi = carry
            base = rr * _W + q * _BLK
            v0 = band[pl.ds(base, 16)]
            v1 = band[pl.ds(base + 12, 16)]
            i0 = rr * _BLK + lanes
            i1 = i0 + 12
            c0 = (v0 > bv) | ((v0 == bv) & (i0 < bi))
            bv = jnp.where(c0, v0, bv)
            bi = jnp.where(c0, i0, bi)
            c1 = (v1 > bv) | ((v1 == bv) & (i1 < bi))
            bv = jnp.where(c1, v1, bv)
            bi = jnp.where(c1, i1, bi)
            return bv, bi

        bv, bi = lax.fori_loop(0, _BLK, rowbody,
                               (jnp.full((16,), -jnp.inf, jnp.float32),
                                jnp.full((16,), 1 << 20, jnp.int32)))
        cv, ci = bv, bi
        for s in (8, 4, 2, 1):
            perm = (lanes + s) & 15
            ov = _lane_gather(cv, perm)
            oi = _lane_gather(ci, perm)
            take = (ov > cv) | ((ov == cv) & (oi < ci))
            cv = jnp.where(take, ov, cv)
            ci = jnp.where(take, oi, ci)
        qq = _fdiv(ci, _BLK)
        r = ci - qq * _BLK
        row = p * _BLK + qq
        col = q * _BLK + r
        rc_l.append(jnp.where(lanes == 0, row.astype(jnp.float32),
                              jnp.where(lanes == 1, col.astype(jnp.float32),
                                        0.0)))
        row0 = row[0]
        col0 = col[0]
        subs = []
        for dh in range(3):
            start = (row0 + dh + 1) * _WP + col0
            start8 = pl.multiple_of((start >> 3) << 3, 8)
            subs.append(start - start8)
            copies.append(pltpu.make_async_copy(
                xt_hbm.at[b, pl.ds(start8, 16)], winv.at[q, dh], sem))
        sub_l.append(subs)
    for c in copies:
        c.start()
    for c in copies:
        c.wait()
    l1 = lanes + 16
    ci0 = _fdiv(lanes, 9)
    rem0 = lanes - 9 * ci0
    dh0 = _fdiv(rem0, 3)
    dw0 = rem0 - 3 * dh0
    ci1 = _fdiv(l1, 9)
    rem1 = l1 - 9 * ci1
    dh1 = _fdiv(rem1, 3)
    dw1 = rem1 - 3 * dh1
    zero = jnp.zeros((16,), jnp.float32)
    for q in range(_P):
        qv = jnp.full((16,), q, jnp.int32)
        s0, s1, s2 = sub_l[q]
        sub0 = jnp.where(dh0 == 0, s0, jnp.where(dh0 == 1, s1, s2))
        sub1 = jnp.where(dh1 == 0, s0, jnp.where(dh1 == 1, s1, s2))
        outv[q, pl.ds(0, 16)] = plsc.load_gather(
            winv, [qv, dh0, sub0 + dw0, ci0])
        outv[q, pl.ds(16, 16)] = plsc.load_gather(
            winv, [qv, dh1, sub1 + dw1, ci1])
        outv[q, pl.ds(32, 16)] = rc_l[q]
        for c in range(3, 8):
            outv[q, pl.ds(c * 16, 16)] = zero
    pltpu.sync_copy(outv, out_hbm.at[pl.ds(wid * _TPS, _TPS)])


def _select_gather(resp3, xtA):
    mesh = plsc.VectorSubcoreMesh(core_axis_name="c", subcore_axis_name="s")
    f = pl.kernel(
        _sel_body,
        mesh=mesh,
        out_type=jax.ShapeDtypeStruct((_B * _K, 128), jnp.float32),
        scratch_types=[
            pltpu.VMEM((_TN,), jnp.float32),
            pltpu.VMEM((_TPS, 3, 16, _CP), jnp.float32),
            pltpu.VMEM((_TPS, 128), jnp.float32),
            pltpu.SemaphoreType.DMA,
        ],
        compiler_params=pltpu.CompilerParams(
            use_tc_tiling_on_sc=True, needs_layout_passes=False),
    )
    return f(resp3, xtA)


def _dist_body(col_ref, pat_ref, rA_ref, cA_ref, wc_ref, wp_ref, brow_ref,
               bcol_ref, dr_ref, dc_ref, mv_ref, desc_s, nA, rmin, ridx):
    nt = pl.program_id(1)

    @pl.when(nt == 0)
    def _():
        d = jnp.dot(pat_ref[0], wp_ref[...], preferred_element_type=jnp.float32)
        d = jnp.maximum(d + brow_ref[...], 0.0)
        desc_s[...] = d
        nA[...] = jnp.sum(d * d, axis=1, keepdims=True)
        rmin[...] = jnp.full((_K, 1), jnp.inf, jnp.float32)
        ridx[...] = jnp.zeros((_K, 1), jnp.int32)

    f = jnp.dot(wc_ref[...], col_ref[0], preferred_element_type=jnp.float32)
    f = jnp.maximum(f + bcol_ref[...], 0.0)
    dots = jnp.dot(desc_s[...], f, preferred_element_type=jnp.float32)
    nb = jnp.dot(jnp.ones((1, _C), jnp.float32), f * f,
                 preferred_element_type=jnp.float32)
    dist = nb - 2.0 * dots
    tmin = jnp.min(dist, axis=1, keepdims=True)
    ii = lax.broadcasted_iota(jnp.int32, (_K, _TN), 1)
    targ = jnp.min(jnp.where(dist == tmin, ii, jnp.int32(_TN)),
                   axis=1, keepdims=True)
    better = tmin < rmin[...]
    ridx[...] = jnp.where(better, targ + nt * _TN, ridx[...])
    rmin[...] = jnp.where(better, tmin, rmin[...])

    @pl.when(nt == _NT - 1)
    def _():
        idx = ridx[...]
        rB = (idx // _W).astype(jnp.float32)
        cB = (idx % _W).astype(jnp.float32)
        dr_ref[0] = rA_ref[0] - rB
        dc_ref[0] = cA_ref[0] - cB
        mv_ref[0] = rmin[...] + nA[...]


def _dist(colB, pats, rA, cA, Wcat, WpatT, brow, bcol):
    out3 = [jax.ShapeDtypeStruct((_B, _K, 1), jnp.float32)] * 3
    return pl.pallas_call(
        _dist_body,
        grid=(_B, _NT),
        in_specs=[
            pl.BlockSpec((1, _KP, _TN), lambda b, n: (b, 0, n)),
            pl.BlockSpec((1, _K, _KP), lambda b, n: (b, 0, 0)),
            pl.BlockSpec((1, _K, 1), lambda b, n: (b, 0, 0)),
            pl.BlockSpec((1, _K, 1), lambda b, n: (b, 0, 0)),
            pl.BlockSpec((_C, _KP), lambda b, n: (0, 0)),
            pl.BlockSpec((_KP, _C), lambda b, n: (0, 0)),
            pl.BlockSpec((1, _C), lambda b, n: (0, 0)),
            pl.BlockSpec((_C, 1), lambda b, n: (0, 0)),
        ],
        out_specs=[pl.BlockSpec((1, _K, 1), lambda b, n: (b, 0, 0))] * 3,
        out_shape=out3,
        scratch_shapes=[
            pltpu.VMEM((_K, _C), jnp.float32),
            pltpu.VMEM((_K, 1), jnp.float32),
            pltpu.VMEM((_K, 1), jnp.float32),
            pltpu.VMEM((_K, 1), jnp.int32),
        ],
        compiler_params=pltpu.CompilerParams(
            dimension_semantics=("arbitrary", "arbitrary")),
    )(colB, pats, rA, cA, Wcat, WpatT, brow, bcol)


@jax.jit
def kernel(xA, xB, Wc, bc):
    colA = _im2col(xA)
    colB = _im2col(xB)
    xtA = _nhwc8(xA)
    Wcat = jnp.pad(Wc.transpose(0, 2, 3, 1).reshape(_C, 27),
                   ((0, 0), (0, _KP - 27)))
    WpatT = jnp.pad(Wc.reshape(_C, 27), ((0, 0), (0, _KP - 27))).T
    brow = bc.reshape(1, _C)
    bcol = bc.reshape(_C, 1)

    resp3 = _resp(colA, Wcat, bcol)
    sel = _select_gather(resp3, xtA)
    po = sel.reshape(_B, _K, 128)
    pats = po[:, :, :_KP]
    rA = po[:, :, 32].reshape(_B, _K, 1)
    cA = po[:, :, 33].reshape(_B, _K, 1)
    dr, dc, mv = _dist(colB, pats, rA, cA, Wcat, WpatT, brow, bcol)
    return jnp.stack([dr[:, :, 0], dc[:, :, 0], mv[:, :, 0]], axis=-1)


# speedup vs baseline: 21.3135x; 1.5976x over previous
import jax
import jax.numpy as jnp
from jax import lax
from jax.experimental import pallas as pl
from jax.experimental.pallas import tpu as pltpu
from jax.experimental.pallas import tpu_sc as plsc

_B = 4
_C = 96
_H = 224
_W = 224
_N = _H * _W
_P = 8
_BLK = 28
_K = 64
_KP = 32
_TN = _BLK * _W
_NT = _H // _BLK
_NB = _B * _NT
_TPS = 8


def _im2col(x):
    xp = jnp.pad(x, ((0, 0), (0, 0), (1, 1), (1, 1)))
    sl = [xp[:, :, dh:dh + _H, dw:dw + _W] for dh in range(3) for dw in range(3)]
    col = jnp.stack(sl, axis=-1)
    col = col.transpose(0, 2, 3, 1, 4).reshape(_B, _N, 27)
    return jnp.pad(col, ((0, 0), (0, 0), (0, _KP - 27)))


def _resp_body(col_ref, wt_ref, b_ref, out_ref):
    x = col_ref[0]
    f = jnp.dot(x, wt_ref[...], preferred_element_type=jnp.float32)
    f = jnp.maximum(f + b_ref[...], 0.0)
    out_ref[0] = lax.dot_general(
        jnp.ones((1, _C), jnp.float32), f,
        dimension_numbers=(((1,), (1,)), ((), ())),
        preferred_element_type=jnp.float32)


def _resp(colA, WfT, brow):
    return pl.pallas_call(
        _resp_body,
        grid=(_NB,),
        in_specs=[
            pl.BlockSpec((1, _TN, _KP), lambda g: (g // _NT, g % _NT, 0)),
            pl.BlockSpec((_KP, _C), lambda g: (0, 0)),
            pl.BlockSpec((1, _C), lambda g: (0, 0)),
        ],
        out_specs=pl.BlockSpec((1, 1, _TN), lambda g: (g, 0, 0)),
        out_shape=jax.ShapeDtypeStruct((_NB, 1, _TN), jnp.float32),
    )(colA, WfT, brow)


def _lane_gather(x, idx):
    dn = lax.GatherDimensionNumbers(
        offset_dims=(), collapsed_slice_dims=(0,), start_index_map=(0,))
    return lax.gather(x, idx[:, None], dn, slice_sizes=(1,),
                      mode=lax.GatherScatterMode.PROMISE_IN_BOUNDS)


def _fdiv(x, d):
    return ((x.astype(jnp.float32) + 0.5) * (1.0 / d)).astype(jnp.int32)


def _sel_body(resp_hbm, col_hbm, out_hbm, band, rows, outv, sem):
    wid = lax.axis_index("s") * 2 + lax.axis_index("c")
    lanes = lax.iota(jnp.int32, 16)
    b = wid >> 3
    p = wid & 7
    pltpu.sync_copy(resp_hbm.at[wid, 0], band)
    rc_l = []
    copies = []
    for q in range(_P):
        def rowbody(rr, carry, q=q):
            bv, bi = carry
            base = rr * _W + q * _BLK
            v0 = band[pl.ds(base, 16)]
            v1 = band[pl.ds(base + 12, 16)]
            i0 = rr * _BLK + lanes
            i1 = i0 + 12
            c0 = (v0 > bv) | ((v0 == bv) & (i0 < bi))
            bv = jnp.where(c0, v0, bv)
            bi = jnp.where(c0, i0, bi)
            c1 = (v1 > bv) | ((v1 == bv) & (i1 < bi))
            bv = jnp.where(c1, v1, bv)
            bi = jnp.where(c1, i1, bi)
            return bv, bi

        bv, bi = lax.fori_loop(0, _BLK, rowbody,
                               (jnp.full((16,), -jnp.inf, jnp.float32),
                                jnp.full((16,), 1 << 20, jnp.int32)))
        cv, ci = bv, bi
        for s in (8, 4, 2, 1):
            perm = (lanes + s) & 15
            ov = _lane_gather(cv, perm)
            oi = _lane_gather(ci, perm)
            take = (ov > cv) | ((ov == cv) & (oi < ci))
            cv = jnp.where(take, ov, cv)
            ci = jnp.where(take, oi, ci)
        qq = _fdiv(ci, _BLK)
        r = ci - qq * _BLK
        row = p * _BLK + qq
        col = q * _BLK + r
        gidx = b * _N + row * _W + col
        rc_l.append(jnp.where(lanes == 0, row.astype(jnp.float32),
                              jnp.where(lanes == 1, col.astype(jnp.float32),
                                        0.0)))
        copies.append(pltpu.make_async_copy(
            col_hbm.at[gidx[0]], rows.at[q], sem))
    for c in copies:
        c.start()
    for c in copies:
        c.wait()
    zero = jnp.zeros((16,), jnp.float32)
    for q in range(_P):
        outv[q, pl.ds(0, 16)] = rows[q, pl.ds(0, 16)]
        outv[q, pl.ds(16, 16)] = rows[q, pl.ds(16, 16)]
        outv[q, pl.ds(32, 16)] = rc_l[q]
        for c in range(3, 8):
            outv[q, pl.ds(c * 16, 16)] = zero
    pltpu.sync_copy(outv, out_hbm.at[pl.ds(wid * _TPS, _TPS)])


def _select_gather(resp3, col_flat):
    mesh = plsc.VectorSubcoreMesh(core_axis_name="c", subcore_axis_name="s")
    f = pl.kernel(
        _sel_body,
        mesh=mesh,
        out_type=jax.ShapeDtypeStruct((_B * _K, 128), jnp.float32),
        scratch_types=[
            pltpu.VMEM((_TN,), jnp.float32),
            pltpu.VMEM((_TPS, _KP), jnp.float32),
            pltpu.VMEM((_TPS, 128), jnp.float32),
            pltpu.SemaphoreType.DMA,
        ],
        compiler_params=pltpu.CompilerParams(
            use_tc_tiling_on_sc=True, needs_layout_passes=False),
    )
    return f(resp3, col_flat)


def _dist_body(col_ref, pT_ref, rA_ref, cA_ref, wt_ref, w96_ref, brow_ref,
               bcol_ref, dr_ref, dc_ref, mv_ref, descT, nA, rmin, ridx):
    nt = pl.program_id(1)

    @pl.when(nt == 0)
    def _():
        d = jnp.dot(w96_ref[...], pT_ref[0], preferred_element_type=jnp.float32)
        d = jnp.maximum(d + bcol_ref[...], 0.0)
        descT[...] = d
        nA[...] = jnp.sum(d * d, axis=0, keepdims=True)
        rmin[...] = jnp.full((1, _K), jnp.inf, jnp.float32)
        ridx[...] = jnp.zeros((1, _K), jnp.int32)

    x = col_ref[0]
    f = jnp.dot(x, wt_ref[...], preferred_element_type=jnp.float32)
    f = jnp.maximum(f + brow_ref[...], 0.0)
    dots = jnp.dot(f, descT[...], preferred_element_type=jnp.float32)
    nb = jnp.sum(f * f, axis=1, keepdims=True)
    dist = nb - 2.0 * dots
    tmin = jnp.min(dist, axis=0, keepdims=True)
    ii = lax.broadcasted_iota(jnp.int32, (_TN, _K), 0)
    targ = jnp.min(jnp.where(dist == tmin, ii, jnp.int32(_TN)),
                   axis=0, keepdims=True)
    better = tmin < rmin[...]
    ridx[...] = jnp.where(better, targ + nt * _TN, ridx[...])
    rmin[...] = jnp.where(better, tmin, rmin[...])

    @pl.when(nt == _NT - 1)
    def _():
        idx = ridx[...]
        rB = (idx // _W).astype(jnp.float32)
        cB = (idx % _W).astype(jnp.float32)
        dr_ref[0] = rA_ref[0] - rB
        dc_ref[0] = cA_ref[0] - cB
        mv_ref[0] = rmin[...] + nA[...]


def _dist(colB, pT, rA, cA, WfT, W96, brow, bcol):
    out3 = [jax.ShapeDtypeStruct((_B, 1, _K), jnp.float32)] * 3
    return pl.pallas_call(
        _dist_body,
        grid=(_B, _NT),
        in_specs=[
            pl.BlockSpec((1, _TN, _KP), lambda b, n: (b, n, 0)),
            pl.BlockSpec((1, _KP, _K), lambda b, n: (b, 0, 0)),
            pl.BlockSpec((1, 1, _K), lambda b, n: (b, 0, 0)),
            pl.BlockSpec((1, 1, _K), lambda b, n: (b, 0, 0)),
            pl.BlockSpec((_KP, _C), lambda b, n: (0, 0)),
            pl.BlockSpec((_C, _KP), lambda b, n: (0, 0)),
            pl.BlockSpec((1, _C), lambda b, n: (0, 0)),
            pl.BlockSpec((_C, 1), lambda b, n: (0, 0)),
        ],
        out_specs=[pl.BlockSpec((1, 1, _K), lambda b, n: (b, 0, 0))] * 3,
        out_shape=out3,
        scratch_shapes=[
            pltpu.VMEM((_C, _K), jnp.float32),
            pltpu.VMEM((1, _K), jnp.float32),
            pltpu.VMEM((1, _K), jnp.float32),
            pltpu.VMEM((1, _K), jnp.int32),
        ],
        compiler_params=pltpu.CompilerParams(
            dimension_semantics=("arbitrary", "arbitrary")),
    )(colB, pT, rA, cA, WfT, W96, brow, bcol)


@jax.jit
def kernel(xA, xB, Wc, bc):
    colA = _im2col(xA)
    colB = _im2col(xB)
    Wf = Wc.reshape(_C, 27)
    W96 = jnp.pad(Wf, ((0, 0), (0, _KP - 27)))
    WfT = W96.T
    brow = bc.reshape(1, _C)
    bcol = bc.reshape(_C, 1)

    resp3 = _resp(colA, WfT, brow)
    sel = _select_gather(resp3, colA.reshape(_B * _N, _KP))
    po = sel.reshape(_B, _K, 128)
    pT = po[:, :, :_KP].transpose(0, 2, 1)
    rA = po[:, :, 32].reshape(_B, 1, _K)
    cA = po[:, :, 33].reshape(_B, 1, _K)
    dr, dc, mv = _dist(colB, pT, rA, cA, WfT, W96, brow, bcol)
    return jnp.stack([dr[:, 0, :], dc[:, 0, :], mv[:, 0, :]], axis=-1)
